# baseline (speedup 1.0000x reference)
import functools

import jax
import jax.numpy as jnp
import numpy as np
from jax import lax
from jax.experimental import pallas as pl
from jax.experimental.pallas import tpu as pltpu
from jax.experimental.pallas import tpu_sc as plsc

FIELD_DIMS = [100000] * 26
NUM_FIELDS = 26
EMBED_DIM = 16
TOTAL_VOCAB = sum(FIELD_DIMS)
EMBED_OUT = NUM_FIELDS * EMBED_DIM
_OFFSETS = np.concatenate(([0], np.cumsum(FIELD_DIMS)[:-1])).astype(np.int32)
BATCH = 4096

_NC = 2
_NS = 16
_NW = _NC * _NS
_CHUNK = 128
_N_PER_W = (BATCH * NUM_FIELDS) // _NW
_NCHUNK = _N_PER_W // _CHUNK


def _sc_gather_body(idx_hbm, emb_hbm, lin_hbm, emb_out, lin_out,
                    idx_v, emb_v, lin_v, sem_e, sem_l):
    wid = lax.axis_index("s") * _NC + lax.axis_index("c")
    base = wid * _N_PER_W

    pltpu.sync_copy(idx_hbm.at[wid], idx_v)

    def _chunk(j, carry):
        idx_row = idx_v.at[j]
        cp_e = pltpu.async_copy(
            emb_hbm.at[idx_row], emb_v.at[pl.ds(j * _CHUNK, _CHUNK)], sem_e)
        cp_l = pltpu.async_copy(
            lin_hbm.at[idx_row], lin_v.at[pl.ds(j * _CHUNK, _CHUNK)], sem_l)
        cp_e.wait()
        cp_l.wait()
        return carry

    lax.fori_loop(0, _NCHUNK, _chunk, 0)

    pltpu.sync_copy(emb_v, emb_out.at[pl.ds(base, _N_PER_W)])
    pltpu.sync_copy(lin_v, lin_out.at[pl.ds(base, _N_PER_W)])


@functools.partial(jax.jit, static_argnums=())
def _sc_gather(idx3, emb_table, lin_table):
    mesh = plsc.VectorSubcoreMesh(core_axis_name="c", subcore_axis_name="s")
    fn = pl.kernel(
        _sc_gather_body,
        out_type=[
            jax.ShapeDtypeStruct((BATCH * NUM_FIELDS, EMBED_DIM), jnp.float32),
            jax.ShapeDtypeStruct((BATCH * NUM_FIELDS, 1), jnp.float32),
        ],
        mesh=mesh,
        compiler_params=pltpu.CompilerParams(use_tc_tiling_on_sc=False),
        scratch_types=[
            pltpu.VMEM((_NCHUNK, _CHUNK), jnp.int32),
            pltpu.VMEM((_N_PER_W, EMBED_DIM), jnp.float32),
            pltpu.VMEM((_N_PER_W, 1), jnp.float32),
            pltpu.SemaphoreType.DMA,
            pltpu.SemaphoreType.DMA,
        ],
    )
    return fn(idx3, emb_table, lin_table)


def _bn_prelu(h, g, be, a):
    mean = jnp.mean(h, axis=0, keepdims=True)
    var = jnp.mean(h * h, axis=0, keepdims=True) - mean * mean
    h = (h - mean) * jax.lax.rsqrt(var + 1e-5) * g + be
    return jnp.maximum(h, 0.0) + a * jnp.minimum(h, 0.0)


def _tc_dense_body(emb_ref, linv_ref, bias_ref, W1_ref, b1_ref, g1_ref, be1_ref,
                   a1_ref, W2_ref, b2_ref, g2_ref, be2_ref, a2_ref, W3_ref,
                   b3_ref, out_ref):
    emb = emb_ref[...]

    row = lax.broadcasted_iota(jnp.int32, (EMBED_OUT, EMBED_DIM), 0)
    col = lax.broadcasted_iota(jnp.int32, (EMBED_OUT, EMBED_DIM), 1)
    S = (row % EMBED_DIM == col).astype(jnp.float32)
    sum_f = jnp.dot(emb, S, preferred_element_type=jnp.float32)
    sum_sq = jnp.dot(emb * emb, S, preferred_element_type=jnp.float32)
    fm = 0.5 * jnp.sum(sum_f * sum_f - sum_sq, axis=1, keepdims=True)

    linear = jnp.sum(linv_ref[...], axis=1, keepdims=True) + bias_ref[0, 0]

    h = jnp.dot(emb, W1_ref[...], preferred_element_type=jnp.float32) + b1_ref[...]
    h = _bn_prelu(h, g1_ref[...], be1_ref[...], a1_ref[0, 0])
    h = jnp.dot(h, W2_ref[...], preferred_element_type=jnp.float32) + b2_ref[...]
    h = _bn_prelu(h, g2_ref[...], be2_ref[...], a2_ref[0, 0])
    mlp = jnp.dot(h, W3_ref[...], preferred_element_type=jnp.float32) + b3_ref[0, 0]

    out_ref[...] = jnp.maximum(linear + fm + mlp, 0.0)


def _tc_dense(emb2, linv, bias, W1, b1, g1, be1, a1, W2, b2, g2, be2, a2, W3, b3):
    return pl.pallas_call(
        _tc_dense_body,
        out_shape=jax.ShapeDtypeStruct((BATCH, 1), jnp.float32),
    )(emb2, linv, bias.reshape(1, 1), W1, b1.reshape(1, -1), g1.reshape(1, -1),
      be1.reshape(1, -1), a1.reshape(1, 1), W2, b2.reshape(1, -1),
      g2.reshape(1, -1), be2.reshape(1, -1), a2.reshape(1, 1), W3,
      b3.reshape(1, 1))


def kernel(x, emb_table, lin_table, bias, W1, b1, g1, be1, a1, W2, b2, g2, be2,
           a2, W3, b3):
    idx = x + jnp.asarray(_OFFSETS)[None, :]
    idx3 = idx.reshape(_NW, _NCHUNK, _CHUNK)
    emb_flat, lin_flat = _sc_gather(idx3, emb_table, lin_table)
    emb2 = emb_flat.reshape(BATCH, EMBED_OUT)
    linv = lin_flat.reshape(BATCH, NUM_FIELDS)
    return _tc_dense(emb2, linv, bias, W1, b1, g1, be1, a1, W2, b2, g2, be2,
                     a2, W3, b3)


# speedup vs baseline: 10.9511x; 10.9511x over previous
import functools

import jax
import jax.numpy as jnp
import numpy as np
from jax import lax
from jax.experimental import pallas as pl
from jax.experimental.pallas import tpu as pltpu
from jax.experimental.pallas import tpu_sc as plsc

NUM_FIELDS = 26
FIELD_DIM = 100000
EMBED_DIM = 16
TOTAL_VOCAB = NUM_FIELDS * FIELD_DIM
EMBED_OUT = NUM_FIELDS * EMBED_DIM
BATCH = 4096

_CW = 6272
_SLABW = 16 * _CW
_NVREG = BATCH // 16
_SLOTS = EMBED_DIM * BATCH
_SLAB_ALLOC = _SLOTS + 128
_FIELDS_PER_SC = NUM_FIELDS // 2
_HITCAP = BATCH + 16


def _sc_body(embT, idx1d, lin1d, out1d, outlin,
             chunk, chunk_tail, idxf, hitvl, hitb, acc_v, acc_p, dump_p,
             linv, shared, sem, semg):
    cid = lax.axis_index("c")
    tid = lax.axis_index("s")
    d16 = lax.iota(jnp.int32, 16)

    for k in range(8):
        dump_p[pl.ds(k * 16, 16)] = d16 + (_SLOTS + k * 16)

    pltpu.sync_copy(embT.at[:, pl.ds(TOTAL_VOCAB - 64, 64)], chunk_tail)

    def field_step(i, carry):
        f = cid * _FIELDS_PER_SC + i
        base_a = f * FIELD_DIM - 32 * (f % 4)
        my_base = pl.multiple_of(base_a + tid * _CW, 128)
        buf = (i % 2) * _SLAB_ALLOC

        pltpu.sync_copy(idx1d.at[pl.ds(f * BATCH, BATCH)], idxf)

        tail = (f == NUM_FIELDS - 1) & (tid == 15)
        cw_eff = jnp.where(tail, 5888, _CW)

        def make_scan(base, width):
            def scan_step(k, off):
                iv = idxf[pl.ds(k * 16, 16)]
                vl = iv - base
                m = (vl >= 0) & (vl < width)
                nh = jnp.sum(m.astype(jnp.int32))

                @pl.when(nh > 0)
                def _():
                    plsc.store_compressed(hitvl.at[pl.ds(off, 16)], vl, mask=m)
                    plsc.store_compressed(hitb.at[pl.ds(off, 16)], k * 16 + d16,
                                          mask=m)

                return off + nh

            return scan_step

        nh_tot = lax.fori_loop(0, _NVREG, make_scan(my_base, cw_eff), 0)

        @pl.when(~tail)
        def _():
            pltpu.sync_copy(embT.at[:, pl.ds(my_base, _CW)], chunk)

        @pl.when(tail)
        def _():
            pltpu.sync_copy(
                embT.at[:, pl.ds(TOTAL_VOCAB - 5952, 5888)],
                chunk.at[:, pl.ds(0, 5888)])

        def reset_acc():
            for k in range(8):
                acc_p[pl.ds(k * 16, 16)] = dump_p[pl.ds(k * 16, 16)] + buf

        def run_hits(src_ref, n_hits):
            reset_acc()
            npad = ((n_hits + 7) // 8) * 8

            def hit_step(j, carry):
                slot = j % 8

                @pl.when(j < n_hits)
                def _():
                    jv = jnp.full((16,), j, jnp.int32)
                    vl_s = plsc.load_gather(hitvl, [jv])
                    b_s = plsc.load_gather(hitb, [jv])
                    vals = plsc.load_gather(src_ref, [d16, vl_s])
                    acc_v[pl.ds(slot * 16, 16)] = vals
                    acc_p[pl.ds(slot * 16, 16)] = d16 * BATCH + b_s + buf

                @pl.when(slot == 7)
                def _():
                    pltpu.async_copy(acc_v, shared.at[acc_p], semg).wait()
                    reset_acc()

                return carry

            lax.fori_loop(0, npad, hit_step, 0)

        run_hits(chunk, nh_tot)

        @pl.when(tail)
        def _():
            n_t = lax.fori_loop(0, _NVREG,
                                make_scan(TOTAL_VOCAB - 64, 64), 0)
            run_hits(chunk_tail, n_t)

        cl0 = pltpu.async_copy(
            lin1d.at[idxf.at[pl.ds(tid * 256, 128)]],
            linv.at[pl.ds(0, 128)], sem)
        cl1 = pltpu.async_copy(
            lin1d.at[idxf.at[pl.ds(tid * 256 + 128, 128)]],
            linv.at[pl.ds(128, 128)], sem)
        cl0.wait()
        cl1.wait()
        pltpu.sync_copy(linv, outlin.at[pl.ds(f * BATCH + tid * 256, 256)])

        plsc.subcore_barrier()
        pltpu.sync_copy(
            shared.at[pl.ds(buf + tid * (_SLOTS // 16), _SLOTS // 16)],
            out1d.at[pl.ds(f * _SLOTS + tid * (_SLOTS // 16), _SLOTS // 16)])
        return carry

    lax.fori_loop(0, _FIELDS_PER_SC, field_step, 0)


def _sc_gather(embT, idx1d, lin1d):
    mesh = plsc.VectorSubcoreMesh(core_axis_name="c", subcore_axis_name="s")
    fn = pl.kernel(
        _sc_body,
        out_type=[
            jax.ShapeDtypeStruct((NUM_FIELDS * _SLOTS,), jnp.float32),
            jax.ShapeDtypeStruct((NUM_FIELDS * BATCH,), jnp.float32),
        ],
        mesh=mesh,
        scratch_types=[
            pltpu.VMEM((16, _CW), jnp.float32),
            pltpu.VMEM((16, 64), jnp.float32),
            pltpu.VMEM((BATCH,), jnp.int32),
            pltpu.VMEM((_HITCAP,), jnp.int32),
            pltpu.VMEM((_HITCAP,), jnp.int32),
            pltpu.VMEM((128,), jnp.float32),
            pltpu.VMEM((128,), jnp.int32),
            pltpu.VMEM((128,), jnp.int32),
            pltpu.VMEM((256,), jnp.float32),
            pltpu.VMEM_SHARED((2 * _SLAB_ALLOC,), jnp.float32),
            pltpu.SemaphoreType.DMA,
            pltpu.SemaphoreType.DMA,
        ],
        compiler_params=pltpu.CompilerParams(use_tc_tiling_on_sc=True,
                                             needs_layout_passes=False),
    )
    return fn(embT, idx1d, lin1d)


def _dotT(a, b):
    return lax.dot_general(a, b, (((0,), (0,)), ((), ())),
                           preferred_element_type=jnp.float32)


def _bn_prelu_T(h, g, be, a):
    mean = jnp.mean(h, axis=1, keepdims=True)
    var = jnp.mean(h * h, axis=1, keepdims=True) - mean * mean
    h = (h - mean) * lax.rsqrt(var + 1e-5) * g + be
    return jnp.maximum(h, 0.0) + a * jnp.minimum(h, 0.0)


def _tc_body(emb_ref, lin_ref, bias_ref, W1_ref, b1_ref, g1_ref, be1_ref,
             a1_ref, W2_ref, b2_ref, g2_ref, be2_ref, a2_ref, W3_ref,
             b3_ref, out_ref):
    emb = emb_ref[...]

    row = lax.broadcasted_iota(jnp.int32, (EMBED_OUT, EMBED_DIM), 0)
    col = lax.broadcasted_iota(jnp.int32, (EMBED_OUT, EMBED_DIM), 1)
    S = (row % EMBED_DIM == col).astype(jnp.float32)

    sum_f = _dotT(S, emb)
    sum_sq = _dotT(S, emb * emb)
    fm = 0.5 * jnp.sum(sum_f * sum_f - sum_sq, axis=0, keepdims=True)

    linear = jnp.sum(lin_ref[...], axis=0, keepdims=True) + bias_ref[0, 0]

    h = _dotT(W1_ref[...], emb) + b1_ref[...]
    h = _bn_prelu_T(h, g1_ref[...], be1_ref[...], a1_ref[0, 0])
    h = _dotT(W2_ref[...], h) + b2_ref[...]
    h = _bn_prelu_T(h, g2_ref[...], be2_ref[...], a2_ref[0, 0])
    mlp = _dotT(W3_ref[...], h) + b3_ref[0, 0]

    out_ref[...] = jnp.maximum(linear + fm + mlp, 0.0)


def _tc_dense(emb2, lin2, bias, W1, b1, g1, be1, a1, W2, b2, g2, be2, a2,
              W3, b3):
    return pl.pallas_call(
        _tc_body,
        out_shape=jax.ShapeDtypeStruct((1, BATCH), jnp.float32),
    )(emb2, lin2, bias.reshape(1, 1), W1, b1.reshape(-1, 1),
      g1.reshape(-1, 1), be1.reshape(-1, 1), a1.reshape(1, 1), W2,
      b2.reshape(-1, 1), g2.reshape(-1, 1), be2.reshape(-1, 1),
      a2.reshape(1, 1), W3, b3.reshape(1, 1))


_OFFSETS = np.arange(NUM_FIELDS, dtype=np.int32) * FIELD_DIM


def kernel(x, emb_table, lin_table, bias, W1, b1, g1, be1, a1, W2, b2, g2,
           be2, a2, W3, b3):
    idx1d = (x + jnp.asarray(_OFFSETS)[None, :]).T.reshape(-1)
    embT = emb_table.T
    lin1d = lin_table.reshape(-1)
    out1d, outlin = _sc_gather(embT, idx1d, lin1d)
    emb2 = out1d.reshape(EMBED_OUT, BATCH)
    lin2 = outlin.reshape(NUM_FIELDS, BATCH)
    res = _tc_dense(emb2, lin2, bias, W1, b1, g1, be1, a1, W2, b2, g2, be2,
                    a2, W3, b3)
    return res.reshape(BATCH, 1)


# speedup vs baseline: 15.6514x; 1.4292x over previous
import functools

import jax
import jax.numpy as jnp
import numpy as np
from jax import lax
from jax.experimental import pallas as pl
from jax.experimental.pallas import tpu as pltpu
from jax.experimental.pallas import tpu_sc as plsc

NUM_FIELDS = 26
FIELD_DIM = 100000
EMBED_DIM = 16
TOTAL_VOCAB = NUM_FIELDS * FIELD_DIM
EMBED_OUT = NUM_FIELDS * EMBED_DIM
BATCH = 4096

_CW = 6272
_SLABW = 16 * _CW
_NVREG = BATCH // 16
_SLOTS = EMBED_DIM * BATCH
_SLAB_ALLOC = _SLOTS + 128
_FIELDS_PER_SC = NUM_FIELDS // 2
_HITCAP = BATCH + 16


def _sc_body(embT, idx1d, lin1d, out1d, outlin,
             chunkA, chunkB, chunk_tail, idxf, hitvlA, hitbA, hitvlB, hitbB,
             acc_v, acc_p2, linv, shared, sem, sem2, semg):
    cid = lax.axis_index("c")
    tid = lax.axis_index("s")
    d16 = lax.iota(jnp.int32, 16)
    _AW = 3200
    _BW = 3072
    _TBW = 2688

    pltpu.sync_copy(embT.at[:, pl.ds(TOTAL_VOCAB - 64, 64)], chunk_tail)

    def field_step(i, carry):
        f = cid * _FIELDS_PER_SC + i
        base_a = f * FIELD_DIM - 32 * (f % 4)
        my_base = pl.multiple_of(base_a + tid * _CW, 128)
        buf = 0

        tail = (f == NUM_FIELDS - 1) & (tid == 15)
        cwB_eff = jnp.where(tail, _TBW, _BW)

        cpA = pltpu.async_copy(embT.at[:, pl.ds(my_base, _AW)], chunkA, sem)

        @pl.when(~tail)
        def _():
            pltpu.async_copy(embT.at[:, pl.ds(my_base + _AW, _BW)],
                             chunkB, sem2)

        @pl.when(tail)
        def _():
            pltpu.async_copy(
                embT.at[:, pl.ds(TOTAL_VOCAB - 64 - _TBW, _TBW)],
                chunkB.at[:, pl.ds(0, _TBW)], sem2)

        pltpu.sync_copy(idx1d.at[pl.ds(f * BATCH, BATCH)], idxf)

        def scan_step(k, offs):
            offA, offB = offs
            iv = idxf[pl.ds(k * 16, 16)]
            vl = iv - my_base
            mA = (vl >= 0) & (vl < _AW)
            vlB = vl - _AW
            mB = (vlB >= 0) & (vlB < cwB_eff)
            nA = jnp.sum(mA.astype(jnp.int32))
            nB = jnp.sum(mB.astype(jnp.int32))
            b_vec = k * 16 + d16

            @pl.when(nA > 0)
            def _():
                plsc.store_compressed(hitvlA.at[pl.ds(offA, 16)], vl, mask=mA)
                plsc.store_compressed(hitbA.at[pl.ds(offA, 16)], b_vec,
                                      mask=mA)

            @pl.when(nB > 0)
            def _():
                plsc.store_compressed(hitvlB.at[pl.ds(offB, 16)], vlB,
                                      mask=mB)
                plsc.store_compressed(hitbB.at[pl.ds(offB, 16)], b_vec,
                                      mask=mB)

            return (offA + nA, offB + nB)

        nA_tot, nB_tot = lax.fori_loop(0, _NVREG, scan_step, (0, 0))

        def run_hits(src_ref, hvl, hb, n_hits):
            hvl[pl.ds(n_hits, 16)] = jnp.zeros((16,), jnp.int32)
            hb[pl.ds(n_hits, 16)] = jnp.full((16,), BATCH, jnp.int32)
            ngrp = (n_hits + 15) // 16

            def grp_step(g, carry):
                vl16 = hvl[pl.ds(g * 16, 16)]
                b16 = hb[pl.ds(g * 16, 16)]
                bad = b16 >= BATCH
                btile = ((b16 >> 7) << 10) + (b16 & 127)
                for d in range(16):
                    vals = plsc.load_gather(src_ref,
                                            [jnp.full((16,), d, jnp.int32),
                                             vl16])
                    cpos = (d // 8) * 32768 + (d % 8) * 128
                    pos = jnp.where(bad, _SLOTS + (d * 16) % 128 + d16,
                                    btile + cpos)
                    acc_v[pl.ds(d * 16, 16)] = vals
                    acc_p2[d // 8, pl.ds((d % 8) * 16, 16)] = pos + buf
                cp0 = pltpu.async_copy(acc_v.at[pl.ds(0, 128)],
                                       shared.at[acc_p2.at[0]], semg)
                cp1 = pltpu.async_copy(acc_v.at[pl.ds(128, 128)],
                                       shared.at[acc_p2.at[1]], semg)
                cp0.wait()
                cp1.wait()
                return carry

            lax.fori_loop(0, ngrp, grp_step, 0)

        cpA.wait()
        run_hits(chunkA, hitvlA, hitbA, nA_tot)

        @pl.when(~tail)
        def _():
            pltpu.make_async_copy(embT.at[:, pl.ds(0, _BW)], chunkB,
                                  sem2).wait()

        @pl.when(tail)
        def _():
            pltpu.make_async_copy(embT.at[:, pl.ds(0, _TBW)],
                                  chunkB.at[:, pl.ds(0, _TBW)], sem2).wait()

        run_hits(chunkB, hitvlB, hitbB, nB_tot)

        @pl.when(tail)
        def _():
            def tail_scan(k, off):
                iv = idxf[pl.ds(k * 16, 16)]
                vt = iv - (TOTAL_VOCAB - 64)
                mt = (vt >= 0) & (vt < 64)
                nt = jnp.sum(mt.astype(jnp.int32))

                @pl.when(nt > 0)
                def _():
                    plsc.store_compressed(hitvlA.at[pl.ds(off, 16)], vt,
                                          mask=mt)
                    plsc.store_compressed(hitbA.at[pl.ds(off, 16)],
                                          k * 16 + d16, mask=mt)

                return off + nt

            n_t = lax.fori_loop(0, _NVREG, tail_scan, 0)
            run_hits(chunk_tail, hitvlA, hitbA, n_t)

        cl0 = pltpu.async_copy(
            lin1d.at[idxf.at[pl.ds(tid * 256, 128)]],
            linv.at[pl.ds(0, 128)], sem)
        cl1 = pltpu.async_copy(
            lin1d.at[idxf.at[pl.ds(tid * 256 + 128, 128)]],
            linv.at[pl.ds(128, 128)], sem)
        cl0.wait()
        cl1.wait()
        pltpu.sync_copy(linv, outlin.at[pl.ds(f * BATCH + tid * 256, 256)])

        plsc.subcore_barrier()
        pltpu.sync_copy(
            shared.at[pl.ds(tid * (_SLOTS // 16), _SLOTS // 16)],
            out1d.at[pl.ds(f * _SLOTS + tid * (_SLOTS // 16), _SLOTS // 16)])
        plsc.subcore_barrier()
        return carry

    lax.fori_loop(0, _FIELDS_PER_SC, field_step, 0)


def _sc_gather(embT, idx1d, lin1d):
    mesh = plsc.VectorSubcoreMesh(core_axis_name="c", subcore_axis_name="s")
    fn = pl.kernel(
        _sc_body,
        out_type=[
            jax.ShapeDtypeStruct((NUM_FIELDS * _SLOTS,), jnp.float32),
            jax.ShapeDtypeStruct((NUM_FIELDS * BATCH,), jnp.float32),
        ],
        mesh=mesh,
        scratch_types=[
            pltpu.VMEM((16, 3200), jnp.float32),
            pltpu.VMEM((16, 3072), jnp.float32),
            pltpu.VMEM((16, 64), jnp.float32),
            pltpu.VMEM((BATCH,), jnp.int32),
            pltpu.VMEM((_HITCAP,), jnp.int32),
            pltpu.VMEM((_HITCAP,), jnp.int32),
            pltpu.VMEM((_HITCAP,), jnp.int32),
            pltpu.VMEM((_HITCAP,), jnp.int32),
            pltpu.VMEM((256,), jnp.float32),
            pltpu.VMEM((2, 128), jnp.int32),
            pltpu.VMEM((256,), jnp.float32),
            pltpu.VMEM_SHARED((_SLAB_ALLOC,), jnp.float32),
            pltpu.SemaphoreType.DMA,
            pltpu.SemaphoreType.DMA,
            pltpu.SemaphoreType.DMA,
        ],
        compiler_params=pltpu.CompilerParams(use_tc_tiling_on_sc=True,
                                             needs_layout_passes=False),
    )
    return fn(embT, idx1d, lin1d)


def _dotT(a, b):
    return lax.dot_general(a, b, (((0,), (0,)), ((), ())),
                           preferred_element_type=jnp.float32)


def _bn_prelu_T(h, g, be, a):
    mean = jnp.mean(h, axis=1, keepdims=True)
    var = jnp.mean(h * h, axis=1, keepdims=True) - mean * mean
    h = (h - mean) * lax.rsqrt(var + 1e-5) * g + be
    return jnp.maximum(h, 0.0) + a * jnp.minimum(h, 0.0)


def _tc_body(emb_ref, lin_ref, bias_ref, W1_ref, b1_ref, g1_ref, be1_ref,
             a1_ref, W2_ref, b2_ref, g2_ref, be2_ref, a2_ref, W3_ref,
             b3_ref, out_ref):
    emb = emb_ref[...]

    row = lax.broadcasted_iota(jnp.int32, (EMBED_OUT, EMBED_DIM), 0)
    col = lax.broadcasted_iota(jnp.int32, (EMBED_OUT, EMBED_DIM), 1)
    S = (row % EMBED_DIM == col).astype(jnp.float32)

    sum_f = _dotT(S, emb)
    sum_sq = _dotT(S, emb * emb)
    fm = 0.5 * jnp.sum(sum_f * sum_f - sum_sq, axis=0, keepdims=True)

    linear = jnp.sum(lin_ref[...], axis=0, keepdims=True) + bias_ref[0, 0]

    h = _dotT(W1_ref[...], emb) + b1_ref[...]
    h = _bn_prelu_T(h, g1_ref[...], be1_ref[...], a1_ref[0, 0])
    h = _dotT(W2_ref[...], h) + b2_ref[...]
    h = _bn_prelu_T(h, g2_ref[...], be2_ref[...], a2_ref[0, 0])
    mlp = _dotT(W3_ref[...], h) + b3_ref[0, 0]

    out_ref[...] = jnp.maximum(linear + fm + mlp, 0.0)


def _tc_dense(emb2, lin2, bias, W1, b1, g1, be1, a1, W2, b2, g2, be2, a2,
              W3, b3):
    return pl.pallas_call(
        _tc_body,
        out_shape=jax.ShapeDtypeStruct((1, BATCH), jnp.float32),
    )(emb2, lin2, bias.reshape(1, 1), W1, b1.reshape(-1, 1),
      g1.reshape(-1, 1), be1.reshape(-1, 1), a1.reshape(1, 1), W2,
      b2.reshape(-1, 1), g2.reshape(-1, 1), be2.reshape(-1, 1),
      a2.reshape(1, 1), W3, b3.reshape(1, 1))


_OFFSETS = np.arange(NUM_FIELDS, dtype=np.int32) * FIELD_DIM


def kernel(x, emb_table, lin_table, bias, W1, b1, g1, be1, a1, W2, b2, g2,
           be2, a2, W3, b3):
    idx1d = (x + jnp.asarray(_OFFSETS)[None, :]).T.reshape(-1)
    embT = emb_table.T
    lin1d = lin_table.reshape(-1)
    out1d, outlin = _sc_gather(embT, idx1d, lin1d)
    emb2 = (out1d.reshape(EMBED_OUT // 8, BATCH // 128, 8, 128)
            .transpose(0, 2, 1, 3).reshape(EMBED_OUT, BATCH))
    lin2 = outlin.reshape(NUM_FIELDS, BATCH)
    res = _tc_dense(emb2, lin2, bias, W1, b1, g1, be1, a1, W2, b2, g2, be2,
                    a2, W3, b3)
    return res.reshape(BATCH, 1)


# speedup vs baseline: 16.3999x; 1.0478x over previous
import functools

import jax
import jax.numpy as jnp
import numpy as np
from jax import lax
from jax.experimental import pallas as pl
from jax.experimental.pallas import tpu as pltpu
from jax.experimental.pallas import tpu_sc as plsc

NUM_FIELDS = 26
FIELD_DIM = 100000
EMBED_DIM = 16
TOTAL_VOCAB = NUM_FIELDS * FIELD_DIM
EMBED_OUT = NUM_FIELDS * EMBED_DIM
BATCH = 4096

_CW = 6272
_SLABW = 16 * _CW
_NVREG = BATCH // 16
_SLOTS = EMBED_DIM * BATCH
_SLAB_ALLOC = _SLOTS + 128
_FIELDS_PER_SC = NUM_FIELDS // 2
_HITCAP = BATCH + 16


def _sc_body(embT, idx1d, lin1d, out1d, outlin,
             chunkA, chunkB, chunk_tail, idxf, hitvlA, hitbA, hitvlB, hitbB,
             acc_v, acc_p2, linv, shared, sem, sem2, semg):
    cid = lax.axis_index("c")
    tid = lax.axis_index("s")
    d16 = lax.iota(jnp.int32, 16)
    _AW = 3200
    _BW = 3072
    _TBW = 2688

    pltpu.sync_copy(embT.at[:, pl.ds(TOTAL_VOCAB - 64, 64)], chunk_tail)

    def field_step(i, carry):
        f = cid * _FIELDS_PER_SC + i
        base_a = f * FIELD_DIM - 32 * (f % 4)
        my_base = pl.multiple_of(base_a + tid * _CW, 128)
        buf = 0

        tail = (f == NUM_FIELDS - 1) & (tid == 15)
        cwB_eff = jnp.where(tail, _TBW, _BW)

        cpA = pltpu.async_copy(embT.at[:, pl.ds(my_base, _AW)], chunkA, sem)

        @pl.when(~tail)
        def _():
            pltpu.async_copy(embT.at[:, pl.ds(my_base + _AW, _BW)],
                             chunkB, sem2)

        @pl.when(tail)
        def _():
            pltpu.async_copy(
                embT.at[:, pl.ds(TOTAL_VOCAB - 64 - _TBW, _TBW)],
                chunkB.at[:, pl.ds(0, _TBW)], sem2)

        pltpu.sync_copy(idx1d.at[pl.ds(f * BATCH, BATCH)], idxf)

        def scan_step(k4, offs):
            offA, offB = offs
            for u in range(4):
                k = k4 * 4 + u
                iv = idxf[pl.ds(k * 16, 16)]
                vl = iv - my_base
                mA = (vl >= 0) & (vl < _AW)
                vlB = vl - _AW
                mB = (vlB >= 0) & (vlB < cwB_eff)
                nA = jnp.sum(mA.astype(jnp.int32))
                nB = jnp.sum(mB.astype(jnp.int32))
                b_vec = k * 16 + d16

                @pl.when(nA > 0)
                def _(offA=offA, vl=vl, mA=mA, b_vec=b_vec):
                    plsc.store_compressed(hitvlA.at[pl.ds(offA, 16)], vl,
                                          mask=mA)
                    plsc.store_compressed(hitbA.at[pl.ds(offA, 16)], b_vec,
                                          mask=mA)

                @pl.when(nB > 0)
                def _(offB=offB, vlB=vlB, mB=mB, b_vec=b_vec):
                    plsc.store_compressed(hitvlB.at[pl.ds(offB, 16)], vlB,
                                          mask=mB)
                    plsc.store_compressed(hitbB.at[pl.ds(offB, 16)], b_vec,
                                          mask=mB)

                offA = offA + nA
                offB = offB + nB
            return (offA, offB)

        nA_tot, nB_tot = lax.fori_loop(0, _NVREG // 4, scan_step, (0, 0))

        def run_hits(src_ref, hvl, hb, n_hits):
            hvl[pl.ds(n_hits, 16)] = jnp.zeros((16,), jnp.int32)
            hb[pl.ds(n_hits, 16)] = jnp.full((16,), BATCH, jnp.int32)
            ngrp = (n_hits + 15) // 16

            def grp_step(g, carry):
                vb = (g % 2) * 256
                rb = (g % 2) * 2
                vl16 = hvl[pl.ds(g * 16, 16)]
                b16 = hb[pl.ds(g * 16, 16)]
                bad = b16 >= BATCH
                btile = ((b16 >> 7) << 10) + (b16 & 127)

                @pl.when(g >= 2)
                def _():
                    pltpu.make_async_copy(acc_v.at[pl.ds(vb, 128)],
                                          shared.at[acc_p2.at[rb]],
                                          semg).wait()
                    pltpu.make_async_copy(acc_v.at[pl.ds(vb + 128, 128)],
                                          shared.at[acc_p2.at[rb + 1]],
                                          semg).wait()

                for d in range(16):
                    vals = plsc.load_gather(src_ref,
                                            [jnp.full((16,), d, jnp.int32),
                                             vl16])
                    cpos = (d // 8) * 32768 + (d % 8) * 128
                    pos = jnp.where(bad, _SLOTS + (d * 16) % 128 + d16,
                                    btile + cpos)
                    acc_v[pl.ds(vb + d * 16, 16)] = vals
                    acc_p2[rb + d // 8, pl.ds((d % 8) * 16, 16)] = pos + buf
                pltpu.async_copy(acc_v.at[pl.ds(vb, 128)],
                                 shared.at[acc_p2.at[rb]], semg)
                pltpu.async_copy(acc_v.at[pl.ds(vb + 128, 128)],
                                 shared.at[acc_p2.at[rb + 1]], semg)
                return carry

            lax.fori_loop(0, ngrp, grp_step, 0)

            def drain_step(g, carry):
                vb = (g % 2) * 256
                rb = (g % 2) * 2
                pltpu.make_async_copy(acc_v.at[pl.ds(vb, 128)],
                                      shared.at[acc_p2.at[rb]], semg).wait()
                pltpu.make_async_copy(acc_v.at[pl.ds(vb + 128, 128)],
                                      shared.at[acc_p2.at[rb + 1]],
                                      semg).wait()
                return carry

            lax.fori_loop(jnp.maximum(ngrp - 2, 0), ngrp, drain_step, 0)

        cpA.wait()
        run_hits(chunkA, hitvlA, hitbA, nA_tot)

        @pl.when(~tail)
        def _():
            pltpu.make_async_copy(embT.at[:, pl.ds(0, _BW)], chunkB,
                                  sem2).wait()

        @pl.when(tail)
        def _():
            pltpu.make_async_copy(embT.at[:, pl.ds(0, _TBW)],
                                  chunkB.at[:, pl.ds(0, _TBW)], sem2).wait()

        run_hits(chunkB, hitvlB, hitbB, nB_tot)

        @pl.when(tail)
        def _():
            def tail_scan(k, off):
                iv = idxf[pl.ds(k * 16, 16)]
                vt = iv - (TOTAL_VOCAB - 64)
                mt = (vt >= 0) & (vt < 64)
                nt = jnp.sum(mt.astype(jnp.int32))

                @pl.when(nt > 0)
                def _():
                    plsc.store_compressed(hitvlA.at[pl.ds(off, 16)], vt,
                                          mask=mt)
                    plsc.store_compressed(hitbA.at[pl.ds(off, 16)],
                                          k * 16 + d16, mask=mt)

                return off + nt

            n_t = lax.fori_loop(0, _NVREG, tail_scan, 0)
            run_hits(chunk_tail, hitvlA, hitbA, n_t)

        cl0 = pltpu.async_copy(
            lin1d.at[idxf.at[pl.ds(tid * 256, 128)]],
            linv.at[pl.ds(0, 128)], sem)
        cl1 = pltpu.async_copy(
            lin1d.at[idxf.at[pl.ds(tid * 256 + 128, 128)]],
            linv.at[pl.ds(128, 128)], sem)
        cl0.wait()
        cl1.wait()
        pltpu.sync_copy(linv, outlin.at[pl.ds(f * BATCH + tid * 256, 256)])

        plsc.subcore_barrier()
        pltpu.sync_copy(
            shared.at[pl.ds(tid * (_SLOTS // 16), _SLOTS // 16)],
            out1d.at[pl.ds(f * _SLOTS + tid * (_SLOTS // 16), _SLOTS // 16)])
        plsc.subcore_barrier()
        return carry

    lax.fori_loop(0, _FIELDS_PER_SC, field_step, 0)


def _sc_gather(embT, idx1d, lin1d):
    mesh = plsc.VectorSubcoreMesh(core_axis_name="c", subcore_axis_name="s")
    fn = pl.kernel(
        _sc_body,
        out_type=[
            jax.ShapeDtypeStruct((NUM_FIELDS * _SLOTS,), jnp.float32),
            jax.ShapeDtypeStruct((NUM_FIELDS * BATCH,), jnp.float32),
        ],
        mesh=mesh,
        scratch_types=[
            pltpu.VMEM((16, 3200), jnp.float32),
            pltpu.VMEM((16, 3072), jnp.float32),
            pltpu.VMEM((16, 64), jnp.float32),
            pltpu.VMEM((BATCH,), jnp.int32),
            pltpu.VMEM((_HITCAP,), jnp.int32),
            pltpu.VMEM((_HITCAP,), jnp.int32),
            pltpu.VMEM((_HITCAP,), jnp.int32),
            pltpu.VMEM((_HITCAP,), jnp.int32),
            pltpu.VMEM((512,), jnp.float32),
            pltpu.VMEM((4, 128), jnp.int32),
            pltpu.VMEM((256,), jnp.float32),
            pltpu.VMEM_SHARED((_SLAB_ALLOC,), jnp.float32),
            pltpu.SemaphoreType.DMA,
            pltpu.SemaphoreType.DMA,
            pltpu.SemaphoreType.DMA,
        ],
        compiler_params=pltpu.CompilerParams(use_tc_tiling_on_sc=True,
                                             needs_layout_passes=False),
    )
    return fn(embT, idx1d, lin1d)


def _dotT(a, b):
    return lax.dot_general(a, b, (((0,), (0,)), ((), ())),
                           preferred_element_type=jnp.float32)


def _bn_prelu_T(h, g, be, a):
    mean = jnp.mean(h, axis=1, keepdims=True)
    var = jnp.mean(h * h, axis=1, keepdims=True) - mean * mean
    h = (h - mean) * lax.rsqrt(var + 1e-5) * g + be
    return jnp.maximum(h, 0.0) + a * jnp.minimum(h, 0.0)


def _tc_body(emb_ref, lin_ref, bias_ref, W1_ref, b1_ref, g1_ref, be1_ref,
             a1_ref, W2_ref, b2_ref, g2_ref, be2_ref, a2_ref, W3_ref,
             b3_ref, out_ref):
    emb = emb_ref[...]

    row = lax.broadcasted_iota(jnp.int32, (EMBED_OUT, EMBED_DIM), 0)
    col = lax.broadcasted_iota(jnp.int32, (EMBED_OUT, EMBED_DIM), 1)
    S = (row % EMBED_DIM == col).astype(jnp.float32)

    sum_f = _dotT(S, emb)
    sum_sq = _dotT(S, emb * emb)
    fm = 0.5 * jnp.sum(sum_f * sum_f - sum_sq, axis=0, keepdims=True)

    linear = jnp.sum(lin_ref[...], axis=0, keepdims=True) + bias_ref[0, 0]

    h = _dotT(W1_ref[...], emb) + b1_ref[...]
    h = _bn_prelu_T(h, g1_ref[...], be1_ref[...], a1_ref[0, 0])
    h = _dotT(W2_ref[...], h) + b2_ref[...]
    h = _bn_prelu_T(h, g2_ref[...], be2_ref[...], a2_ref[0, 0])
    mlp = _dotT(W3_ref[...], h) + b3_ref[0, 0]

    out_ref[...] = jnp.maximum(linear + fm + mlp, 0.0)


def _tc_dense(emb2, lin2, bias, W1, b1, g1, be1, a1, W2, b2, g2, be2, a2,
              W3, b3):
    return pl.pallas_call(
        _tc_body,
        out_shape=jax.ShapeDtypeStruct((1, BATCH), jnp.float32),
    )(emb2, lin2, bias.reshape(1, 1), W1, b1.reshape(-1, 1),
      g1.reshape(-1, 1), be1.reshape(-1, 1), a1.reshape(1, 1), W2,
      b2.reshape(-1, 1), g2.reshape(-1, 1), be2.reshape(-1, 1),
      a2.reshape(1, 1), W3, b3.reshape(1, 1))


_OFFSETS = np.arange(NUM_FIELDS, dtype=np.int32) * FIELD_DIM


def kernel(x, emb_table, lin_table, bias, W1, b1, g1, be1, a1, W2, b2, g2,
           be2, a2, W3, b3):
    idx1d = (x + jnp.asarray(_OFFSETS)[None, :]).T.reshape(-1)
    embT = emb_table.T
    lin1d = lin_table.reshape(-1)
    out1d, outlin = _sc_gather(embT, idx1d, lin1d)
    emb2 = (out1d.reshape(EMBED_OUT // 8, BATCH // 128, 8, 128)
            .transpose(0, 2, 1, 3).reshape(EMBED_OUT, BATCH))
    lin2 = outlin.reshape(NUM_FIELDS, BATCH)
    res = _tc_dense(emb2, lin2, bias, W1, b1, g1, be1, a1, W2, b2, g2, be2,
                    a2, W3, b3)
    return res.reshape(BATCH, 1)


# speedup vs baseline: 16.6368x; 1.0145x over previous
import functools

import jax
import jax.numpy as jnp
import numpy as np
from jax import lax
from jax.experimental import pallas as pl
from jax.experimental.pallas import tpu as pltpu
from jax.experimental.pallas import tpu_sc as plsc

NUM_FIELDS = 26
FIELD_DIM = 100000
EMBED_DIM = 16
TOTAL_VOCAB = NUM_FIELDS * FIELD_DIM
EMBED_OUT = NUM_FIELDS * EMBED_DIM
BATCH = 4096

_CW = 6272
_SLABW = 16 * _CW
_NVREG = BATCH // 16
_SLOTS = EMBED_DIM * BATCH
_SLAB_ALLOC = _SLOTS + 128
_FIELDS_PER_SC = NUM_FIELDS // 2
_HITCAP = BATCH + 16


def _sc_body(embT, idx1d, lin1d, out1d, outlin,
             chunkA, chunkB, chunk_tail, idxf, hitvlA, hitbA,
             acc_v, acc_p2, linv, shared, sem, sem2, semg):
    cid = lax.axis_index("c")
    tid = lax.axis_index("s")
    d16 = lax.iota(jnp.int32, 16)
    _AW = 3200
    _BW = 3072
    _TBW = 2688

    pltpu.sync_copy(embT.at[:, pl.ds(TOTAL_VOCAB - 64, 64)], chunk_tail)

    def field_step(i, carry):
        f = cid * _FIELDS_PER_SC + i
        base_a = f * FIELD_DIM - 32 * (f % 4)
        my_base = pl.multiple_of(base_a + tid * _CW, 128)
        buf = 0

        tail = (f == NUM_FIELDS - 1) & (tid == 15)
        cw_eff = jnp.where(tail, _AW + _TBW, _CW)

        cpA = pltpu.async_copy(embT.at[:, pl.ds(my_base, _AW)], chunkA, sem)

        @pl.when(~tail)
        def _():
            pltpu.async_copy(embT.at[:, pl.ds(my_base + _AW, _BW)],
                             chunkB, sem2)

        @pl.when(tail)
        def _():
            pltpu.async_copy(
                embT.at[:, pl.ds(TOTAL_VOCAB - 64 - _TBW, _TBW)],
                chunkB.at[:, pl.ds(0, _TBW)], sem2)

        pltpu.sync_copy(idx1d.at[pl.ds(f * BATCH, BATCH)], idxf)

        def scan_step(k4, off):
            for u in range(4):
                k = k4 * 4 + u
                iv = idxf[pl.ds(k * 16, 16)]
                vl = iv - my_base
                m = (vl >= 0) & (vl < cw_eff)
                nh = jnp.sum(m.astype(jnp.int32))
                b_vec = k * 16 + d16

                @pl.when(nh > 0)
                def _(off=off, vl=vl, m=m, b_vec=b_vec):
                    plsc.store_compressed(hitvlA.at[pl.ds(off, 16)], vl,
                                          mask=m)
                    plsc.store_compressed(hitbA.at[pl.ds(off, 16)], b_vec,
                                          mask=m)

                off = off + nh
            return off

        n_tot = lax.fori_loop(0, _NVREG // 4, scan_step, 0)

        def run_hits(srcA, srcB, hvl, hb, n_hits):
            hvl[pl.ds(n_hits, 16)] = jnp.zeros((16,), jnp.int32)
            hb[pl.ds(n_hits, 16)] = jnp.full((16,), BATCH, jnp.int32)
            ngrp = (n_hits + 15) // 16

            def grp_step(g, carry):
                vb = (g % 2) * 256
                rb = (g % 2) * 2
                vl16 = hvl[pl.ds(g * 16, 16)]
                b16 = hb[pl.ds(g * 16, 16)]
                bad = b16 >= BATCH
                inA = vl16 < _AW
                vlA = jnp.minimum(vl16, _AW - 1)
                vlB = jnp.maximum(vl16 - _AW, 0)
                btile = ((b16 >> 7) << 10) + (b16 & 127)

                @pl.when(g >= 2)
                def _():
                    pltpu.make_async_copy(acc_v.at[pl.ds(vb, 128)],
                                          shared.at[acc_p2.at[rb]],
                                          semg).wait()
                    pltpu.make_async_copy(acc_v.at[pl.ds(vb + 128, 128)],
                                          shared.at[acc_p2.at[rb + 1]],
                                          semg).wait()

                for d in range(16):
                    dv = jnp.full((16,), d, jnp.int32)
                    valsA = plsc.load_gather(srcA, [dv, vlA])
                    valsB = plsc.load_gather(srcB, [dv, vlB])
                    vals = jnp.where(inA, valsA, valsB)
                    cpos = (d // 8) * 32768 + (d % 8) * 128
                    pos = jnp.where(bad, _SLOTS + (d * 16) % 128 + d16,
                                    btile + cpos)
                    acc_v[pl.ds(vb + d * 16, 16)] = vals
                    acc_p2[rb + d // 8, pl.ds((d % 8) * 16, 16)] = pos + buf
                pltpu.async_copy(acc_v.at[pl.ds(vb, 128)],
                                 shared.at[acc_p2.at[rb]], semg)
                pltpu.async_copy(acc_v.at[pl.ds(vb + 128, 128)],
                                 shared.at[acc_p2.at[rb + 1]], semg)
                return carry

            lax.fori_loop(0, ngrp, grp_step, 0)

            def drain_step(g, carry):
                vb = (g % 2) * 256
                rb = (g % 2) * 2
                pltpu.make_async_copy(acc_v.at[pl.ds(vb, 128)],
                                      shared.at[acc_p2.at[rb]], semg).wait()
                pltpu.make_async_copy(acc_v.at[pl.ds(vb + 128, 128)],
                                      shared.at[acc_p2.at[rb + 1]],
                                      semg).wait()
                return carry

            lax.fori_loop(jnp.maximum(ngrp - 2, 0), ngrp, drain_step, 0)

        cpA.wait()

        @pl.when(~tail)
        def _():
            pltpu.make_async_copy(embT.at[:, pl.ds(0, _BW)], chunkB,
                                  sem2).wait()

        @pl.when(tail)
        def _():
            pltpu.make_async_copy(embT.at[:, pl.ds(0, _TBW)],
                                  chunkB.at[:, pl.ds(0, _TBW)], sem2).wait()

        run_hits(chunkA, chunkB, hitvlA, hitbA, n_tot)

        @pl.when(tail)
        def _():
            def tail_scan(k, off):
                iv = idxf[pl.ds(k * 16, 16)]
                vt = iv - (TOTAL_VOCAB - 64)
                mt = (vt >= 0) & (vt < 64)
                nt = jnp.sum(mt.astype(jnp.int32))

                @pl.when(nt > 0)
                def _():
                    plsc.store_compressed(hitvlA.at[pl.ds(off, 16)], vt,
                                          mask=mt)
                    plsc.store_compressed(hitbA.at[pl.ds(off, 16)],
                                          k * 16 + d16, mask=mt)

                return off + nt

            n_t = lax.fori_loop(0, _NVREG, tail_scan, 0)
            run_hits(chunk_tail, chunk_tail, hitvlA, hitbA, n_t)

        cl0 = pltpu.async_copy(
            lin1d.at[idxf.at[pl.ds(tid * 256, 128)]],
            linv.at[pl.ds(0, 128)], sem)
        cl1 = pltpu.async_copy(
            lin1d.at[idxf.at[pl.ds(tid * 256 + 128, 128)]],
            linv.at[pl.ds(128, 128)], sem)
        cl0.wait()
        cl1.wait()
        pltpu.sync_copy(linv, outlin.at[pl.ds(f * BATCH + tid * 256, 256)])

        plsc.subcore_barrier()
        pltpu.sync_copy(
            shared.at[pl.ds(tid * (_SLOTS // 16), _SLOTS // 16)],
            out1d.at[pl.ds(f * _SLOTS + tid * (_SLOTS // 16), _SLOTS // 16)])
        plsc.subcore_barrier()
        return carry

    lax.fori_loop(0, _FIELDS_PER_SC, field_step, 0)


def _sc_gather(embT, idx1d, lin1d):
    mesh = plsc.VectorSubcoreMesh(core_axis_name="c", subcore_axis_name="s")
    fn = pl.kernel(
        _sc_body,
        out_type=[
            jax.ShapeDtypeStruct((NUM_FIELDS * _SLOTS,), jnp.float32),
            jax.ShapeDtypeStruct((NUM_FIELDS * BATCH,), jnp.float32),
        ],
        mesh=mesh,
        scratch_types=[
            pltpu.VMEM((16, 3200), jnp.float32),
            pltpu.VMEM((16, 3072), jnp.float32),
            pltpu.VMEM((16, 64), jnp.float32),
            pltpu.VMEM((BATCH,), jnp.int32),
            pltpu.VMEM((_HITCAP,), jnp.int32),
            pltpu.VMEM((_HITCAP,), jnp.int32),
            pltpu.VMEM((512,), jnp.float32),
            pltpu.VMEM((4, 128), jnp.int32),
            pltpu.VMEM((256,), jnp.float32),
            pltpu.VMEM_SHARED((_SLAB_ALLOC,), jnp.float32),
            pltpu.SemaphoreType.DMA,
            pltpu.SemaphoreType.DMA,
            pltpu.SemaphoreType.DMA,
        ],
        compiler_params=pltpu.CompilerParams(use_tc_tiling_on_sc=True,
                                             needs_layout_passes=False),
    )
    return fn(embT, idx1d, lin1d)


def _dotT(a, b):
    return lax.dot_general(a, b, (((0,), (0,)), ((), ())),
                           preferred_element_type=jnp.float32)


def _bn_prelu_T(h, g, be, a):
    mean = jnp.mean(h, axis=1, keepdims=True)
    var = jnp.mean(h * h, axis=1, keepdims=True) - mean * mean
    h = (h - mean) * lax.rsqrt(var + 1e-5) * g + be
    return jnp.maximum(h, 0.0) + a * jnp.minimum(h, 0.0)


def _tc_body(emb_ref, lin_ref, bias_ref, W1_ref, b1_ref, g1_ref, be1_ref,
             a1_ref, W2_ref, b2_ref, g2_ref, be2_ref, a2_ref, W3_ref,
             b3_ref, out_ref):
    emb = emb_ref[...]

    row = lax.broadcasted_iota(jnp.int32, (EMBED_OUT, EMBED_DIM), 0)
    col = lax.broadcasted_iota(jnp.int32, (EMBED_OUT, EMBED_DIM), 1)
    S = (row % EMBED_DIM == col).astype(jnp.float32)

    sum_f = _dotT(S, emb)
    sum_sq = _dotT(S, emb * emb)
    fm = 0.5 * jnp.sum(sum_f * sum_f - sum_sq, axis=0, keepdims=True)

    linear = jnp.sum(lin_ref[...], axis=0, keepdims=True) + bias_ref[0, 0]

    h = _dotT(W1_ref[...], emb) + b1_ref[...]
    h = _bn_prelu_T(h, g1_ref[...], be1_ref[...], a1_ref[0, 0])
    h = _dotT(W2_ref[...], h) + b2_ref[...]
    h = _bn_prelu_T(h, g2_ref[...], be2_ref[...], a2_ref[0, 0])
    mlp = _dotT(W3_ref[...], h) + b3_ref[0, 0]

    out_ref[...] = jnp.maximum(linear + fm + mlp, 0.0)


def _tc_dense(emb2, lin2, bias, W1, b1, g1, be1, a1, W2, b2, g2, be2, a2,
              W3, b3):
    return pl.pallas_call(
        _tc_body,
        out_shape=jax.ShapeDtypeStruct((1, BATCH), jnp.float32),
    )(emb2, lin2, bias.reshape(1, 1), W1, b1.reshape(-1, 1),
      g1.reshape(-1, 1), be1.reshape(-1, 1), a1.reshape(1, 1), W2,
      b2.reshape(-1, 1), g2.reshape(-1, 1), be2.reshape(-1, 1),
      a2.reshape(1, 1), W3, b3.reshape(1, 1))


_OFFSETS = np.arange(NUM_FIELDS, dtype=np.int32) * FIELD_DIM


def kernel(x, emb_table, lin_table, bias, W1, b1, g1, be1, a1, W2, b2, g2,
           be2, a2, W3, b3):
    idx1d = (x + jnp.asarray(_OFFSETS)[None, :]).T.reshape(-1)
    embT = emb_table.T
    lin1d = lin_table.reshape(-1)
    out1d, outlin = _sc_gather(embT, idx1d, lin1d)
    emb2 = (out1d.reshape(EMBED_OUT // 8, BATCH // 128, 8, 128)
            .transpose(0, 2, 1, 3).reshape(EMBED_OUT, BATCH))
    lin2 = outlin.reshape(NUM_FIELDS, BATCH)
    res = _tc_dense(emb2, lin2, bias, W1, b1, g1, be1, a1, W2, b2, g2, be2,
                    a2, W3, b3)
    return res.reshape(BATCH, 1)


# speedup vs baseline: 17.1297x; 1.0296x over previous
import functools

import jax
import jax.numpy as jnp
import numpy as np
from jax import lax
from jax.experimental import pallas as pl
from jax.experimental.pallas import tpu as pltpu
from jax.experimental.pallas import tpu_sc as plsc

NUM_FIELDS = 26
FIELD_DIM = 100000
EMBED_DIM = 16
TOTAL_VOCAB = NUM_FIELDS * FIELD_DIM
EMBED_OUT = NUM_FIELDS * EMBED_DIM
BATCH = 4096

_CW = 6272
_SLABW = 16 * _CW
_NVREG = BATCH // 16
_SLOTS = EMBED_DIM * BATCH
_SLAB_ALLOC = _SLOTS + 128
_FIELDS_PER_SC = NUM_FIELDS // 2
_HITCAP = BATCH + 16


def _sc_body(embT, idx1d, out1d,
             chunkA, chunkB, chunk_tail, idxf, hitvlA, hitbA,
             acc_v, acc_p2, shared, sem, sem2, semg):
    cid = lax.axis_index("c")
    tid = lax.axis_index("s")
    d16 = lax.iota(jnp.int32, 16)
    _AW = 3200
    _BW = 3072
    _TBW = 2688

    pltpu.sync_copy(embT.at[:, pl.ds(TOTAL_VOCAB - 64, 64)], chunk_tail)

    def field_step(i, carry):
        f = cid * _FIELDS_PER_SC + i
        base_a = f * FIELD_DIM - 32 * (f % 4)
        my_base = pl.multiple_of(base_a + tid * _CW, 128)
        buf = 0

        tail = (f == NUM_FIELDS - 1) & (tid == 15)
        cw_eff = jnp.where(tail, _AW + _TBW, _CW)

        cpA = pltpu.async_copy(embT.at[:, pl.ds(my_base, _AW)], chunkA, sem)

        @pl.when(~tail)
        def _():
            pltpu.async_copy(embT.at[:, pl.ds(my_base + _AW, _BW)],
                             chunkB, sem2)

        @pl.when(tail)
        def _():
            pltpu.async_copy(
                embT.at[:, pl.ds(TOTAL_VOCAB - 64 - _TBW, _TBW)],
                chunkB.at[:, pl.ds(0, _TBW)], sem2)

        pltpu.sync_copy(idx1d.at[pl.ds(f * BATCH, BATCH)], idxf)

        def scan_step(k4, off):
            for u in range(4):
                k = k4 * 4 + u
                iv = idxf[pl.ds(k * 16, 16)]
                vl = iv - my_base
                m = (vl >= 0) & (vl < cw_eff)
                nh = jnp.sum(m.astype(jnp.int32))
                b_vec = k * 16 + d16

                @pl.when(nh > 0)
                def _(off=off, vl=vl, m=m, b_vec=b_vec):
                    plsc.store_compressed(hitvlA.at[pl.ds(off, 16)], vl,
                                          mask=m)
                    plsc.store_compressed(hitbA.at[pl.ds(off, 16)], b_vec,
                                          mask=m)

                off = off + nh
            return off

        n_tot = lax.fori_loop(0, _NVREG // 4, scan_step, 0)

        def run_hits(srcA, srcB, hvl, hb, n_hits):
            hvl[pl.ds(n_hits, 16)] = jnp.zeros((16,), jnp.int32)
            hb[pl.ds(n_hits, 16)] = jnp.full((16,), BATCH, jnp.int32)
            ngrp = (n_hits + 15) // 16

            def grp_step(g, carry):
                vb = (g % 2) * 256
                rb = (g % 2) * 2
                vl16 = hvl[pl.ds(g * 16, 16)]
                b16 = hb[pl.ds(g * 16, 16)]
                bad = b16 >= BATCH
                inA = vl16 < _AW
                vlA = jnp.minimum(vl16, _AW - 1)
                vlB = jnp.maximum(vl16 - _AW, 0)
                btile = ((b16 >> 7) << 10) + (b16 & 127)

                @pl.when(g >= 2)
                def _():
                    pltpu.make_async_copy(acc_v.at[pl.ds(vb, 128)],
                                          shared.at[acc_p2.at[rb]],
                                          semg).wait()
                    pltpu.make_async_copy(acc_v.at[pl.ds(vb + 128, 128)],
                                          shared.at[acc_p2.at[rb + 1]],
                                          semg).wait()

                for d in range(16):
                    dv = jnp.full((16,), d, jnp.int32)
                    valsA = plsc.load_gather(srcA, [dv, vlA])
                    valsB = plsc.load_gather(srcB, [dv, vlB])
                    vals = jnp.where(inA, valsA, valsB)
                    cpos = (d // 8) * 32768 + (d % 8) * 128
                    pos = jnp.where(bad, _SLOTS + (d * 16) % 128 + d16,
                                    btile + cpos)
                    acc_v[pl.ds(vb + d * 16, 16)] = vals
                    acc_p2[rb + d // 8, pl.ds((d % 8) * 16, 16)] = pos + buf
                pltpu.async_copy(acc_v.at[pl.ds(vb, 128)],
                                 shared.at[acc_p2.at[rb]], semg)
                pltpu.async_copy(acc_v.at[pl.ds(vb + 128, 128)],
                                 shared.at[acc_p2.at[rb + 1]], semg)
                return carry

            lax.fori_loop(0, ngrp, grp_step, 0)

            def drain_step(g, carry):
                vb = (g % 2) * 256
                rb = (g % 2) * 2
                pltpu.make_async_copy(acc_v.at[pl.ds(vb, 128)],
                                      shared.at[acc_p2.at[rb]], semg).wait()
                pltpu.make_async_copy(acc_v.at[pl.ds(vb + 128, 128)],
                                      shared.at[acc_p2.at[rb + 1]],
                                      semg).wait()
                return carry

            lax.fori_loop(jnp.maximum(ngrp - 2, 0), ngrp, drain_step, 0)

        cpA.wait()

        @pl.when(~tail)
        def _():
            pltpu.make_async_copy(embT.at[:, pl.ds(0, _BW)], chunkB,
                                  sem2).wait()

        @pl.when(tail)
        def _():
            pltpu.make_async_copy(embT.at[:, pl.ds(0, _TBW)],
                                  chunkB.at[:, pl.ds(0, _TBW)], sem2).wait()

        run_hits(chunkA, chunkB, hitvlA, hitbA, n_tot)

        @pl.when(tail)
        def _():
            def tail_scan(k, off):
                iv = idxf[pl.ds(k * 16, 16)]
                vt = iv - (TOTAL_VOCAB - 64)
                mt = (vt >= 0) & (vt < 64)
                nt = jnp.sum(mt.astype(jnp.int32))

                @pl.when(nt > 0)
                def _():
                    plsc.store_compressed(hitvlA.at[pl.ds(off, 16)], vt,
                                          mask=mt)
                    plsc.store_compressed(hitbA.at[pl.ds(off, 16)],
                                          k * 16 + d16, mask=mt)

                return off + nt

            n_t = lax.fori_loop(0, _NVREG, tail_scan, 0)
            run_hits(chunk_tail, chunk_tail, hitvlA, hitbA, n_t)

        plsc.subcore_barrier()
        pltpu.sync_copy(
            shared.at[pl.ds(tid * (_SLOTS // 16), _SLOTS // 16)],
            out1d.at[pl.ds(f * _SLOTS + tid * (_SLOTS // 16), _SLOTS // 16)])
        plsc.subcore_barrier()
        return carry

    lax.fori_loop(0, _FIELDS_PER_SC, field_step, 0)


def _sc_gather(embT, idx1d):
    mesh = plsc.VectorSubcoreMesh(core_axis_name="c", subcore_axis_name="s")
    fn = pl.kernel(
        _sc_body,
        out_type=[
            jax.ShapeDtypeStruct((NUM_FIELDS * _SLOTS,), jnp.float32),
        ],
        mesh=mesh,
        scratch_types=[
            pltpu.VMEM((16, 3200), jnp.float32),
            pltpu.VMEM((16, 3072), jnp.float32),
            pltpu.VMEM((16, 64), jnp.float32),
            pltpu.VMEM((BATCH,), jnp.int32),
            pltpu.VMEM((_HITCAP,), jnp.int32),
            pltpu.VMEM((_HITCAP,), jnp.int32),
            pltpu.VMEM((512,), jnp.float32),
            pltpu.VMEM((4, 128), jnp.int32),
            pltpu.VMEM_SHARED((_SLAB_ALLOC,), jnp.float32),
            pltpu.SemaphoreType.DMA,
            pltpu.SemaphoreType.DMA,
            pltpu.SemaphoreType.DMA,
        ],
        compiler_params=pltpu.CompilerParams(use_tc_tiling_on_sc=True,
                                             needs_layout_passes=False),
    )
    return fn(embT, idx1d)


_NPW = (BATCH * NUM_FIELDS) // 32


def _sc_lin_body(idx1d, lin1d, outlin, idxw, linw, sem):
    wid = lax.axis_index("s") * 2 + lax.axis_index("c")
    base = wid * _NPW
    pltpu.sync_copy(idx1d.at[pl.ds(base, _NPW)], idxw)

    def fire(j, carry):
        pltpu.async_copy(lin1d.at[idxw.at[pl.ds(j * 128, 128)]],
                         linw.at[pl.ds(j * 128, 128)], sem)
        return carry

    lax.fori_loop(0, _NPW // 128, fire, 0)

    def drain(j, carry):
        pltpu.make_async_copy(lin1d.at[pl.ds(0, 128)],
                              linw.at[pl.ds(j * 128, 128)], sem).wait()
        return carry

    lax.fori_loop(0, _NPW // 128, drain, 0)
    pltpu.sync_copy(linw, outlin.at[pl.ds(base, _NPW)])


def _sc_lin(idx1d, lin1d):
    mesh = plsc.VectorSubcoreMesh(core_axis_name="c", subcore_axis_name="s")
    fn = pl.kernel(
        _sc_lin_body,
        out_type=[jax.ShapeDtypeStruct((NUM_FIELDS * BATCH,), jnp.float32)],
        mesh=mesh,
        scratch_types=[
            pltpu.VMEM((_NPW,), jnp.int32),
            pltpu.VMEM((_NPW,), jnp.float32),
            pltpu.SemaphoreType.DMA,
        ],
        compiler_params=pltpu.CompilerParams(use_tc_tiling_on_sc=True,
                                             needs_layout_passes=False),
    )
    return fn(idx1d, lin1d)


def _dotT(a, b):
    return lax.dot_general(a, b, (((0,), (0,)), ((), ())),
                           preferred_element_type=jnp.float32)


def _bn_prelu_T(h, g, be, a):
    mean = jnp.mean(h, axis=1, keepdims=True)
    var = jnp.mean(h * h, axis=1, keepdims=True) - mean * mean
    h = (h - mean) * lax.rsqrt(var + 1e-5) * g + be
    return jnp.maximum(h, 0.0) + a * jnp.minimum(h, 0.0)


def _tc_body(emb_ref, lin_ref, bias_ref, W1_ref, b1_ref, g1_ref, be1_ref,
             a1_ref, W2_ref, b2_ref, g2_ref, be2_ref, a2_ref, W3_ref,
             b3_ref, out_ref):
    emb = emb_ref[...]

    row = lax.broadcasted_iota(jnp.int32, (EMBED_OUT, EMBED_DIM), 0)
    col = lax.broadcasted_iota(jnp.int32, (EMBED_OUT, EMBED_DIM), 1)
    S = (row % EMBED_DIM == col).astype(jnp.float32)

    sum_f = _dotT(S, emb)
    sum_sq = _dotT(S, emb * emb)
    fm = 0.5 * jnp.sum(sum_f * sum_f - sum_sq, axis=0, keepdims=True)

    linear = jnp.sum(lin_ref[...], axis=0, keepdims=True) + bias_ref[0, 0]

    h = _dotT(W1_ref[...], emb) + b1_ref[...]
    h = _bn_prelu_T(h, g1_ref[...], be1_ref[...], a1_ref[0, 0])
    h = _dotT(W2_ref[...], h) + b2_ref[...]
    h = _bn_prelu_T(h, g2_ref[...], be2_ref[...], a2_ref[0, 0])
    mlp = _dotT(W3_ref[...], h) + b3_ref[0, 0]

    out_ref[...] = jnp.maximum(linear + fm + mlp, 0.0)


def _tc_dense(emb2, lin2, bias, W1, b1, g1, be1, a1, W2, b2, g2, be2, a2,
              W3, b3):
    return pl.pallas_call(
        _tc_body,
        out_shape=jax.ShapeDtypeStruct((1, BATCH), jnp.float32),
    )(emb2, lin2, bias.reshape(1, 1), W1, b1.reshape(-1, 1),
      g1.reshape(-1, 1), be1.reshape(-1, 1), a1.reshape(1, 1), W2,
      b2.reshape(-1, 1), g2.reshape(-1, 1), be2.reshape(-1, 1),
      a2.reshape(1, 1), W3, b3.reshape(1, 1))


_OFFSETS = np.arange(NUM_FIELDS, dtype=np.int32) * FIELD_DIM


def kernel(x, emb_table, lin_table, bias, W1, b1, g1, be1, a1, W2, b2, g2,
           be2, a2, W3, b3):
    idx1d = (x + jnp.asarray(_OFFSETS)[None, :]).T.reshape(-1)
    embT = emb_table.T
    lin1d = lin_table.T.reshape(-1)
    out1d, = _sc_gather(embT, idx1d)
    outlin, = _sc_lin(idx1d, lin1d)
    emb2 = (out1d.reshape(EMBED_OUT // 8, BATCH // 128, 8, 128)
            .transpose(0, 2, 1, 3).reshape(EMBED_OUT, BATCH))
    lin2 = outlin.reshape(NUM_FIELDS, BATCH)
    res = _tc_dense(emb2, lin2, bias, W1, b1, g1, be1, a1, W2, b2, g2, be2,
                    a2, W3, b3)
    return res.reshape(BATCH, 1)


# speedup vs baseline: 17.1555x; 1.0015x over previous
import functools

import jax
import jax.numpy as jnp
import numpy as np
from jax import lax
from jax.experimental import pallas as pl
from jax.experimental.pallas import tpu as pltpu
from jax.experimental.pallas import tpu_sc as plsc

NUM_FIELDS = 26
FIELD_DIM = 100000
EMBED_DIM = 16
TOTAL_VOCAB = NUM_FIELDS * FIELD_DIM
EMBED_OUT = NUM_FIELDS * EMBED_DIM
BATCH = 4096

_CW = 6272
_SLABW = 16 * _CW
_NVREG = BATCH // 16
_SLOTS = EMBED_DIM * BATCH
_SLAB_ALLOC = _SLOTS + 128
_FIELDS_PER_SC = NUM_FIELDS // 2
_HITCAP = BATCH + 16


def _sc_body(embT, idx1d, out1d,
             chunkA, chunkB, chunk_tail, idxf, hitvlA, hitbA,
             acc_v, acc_p2, shared, sem, sem2, semg):
    cid = lax.axis_index("c")
    tid = lax.axis_index("s")
    d16 = lax.iota(jnp.int32, 16)
    _AW = 3200
    _BW = 3072
    _TBW = 2688

    pltpu.sync_copy(embT.at[:, pl.ds(TOTAL_VOCAB - 64, 64)], chunk_tail)

    def field_step(i, carry):
        f = cid * _FIELDS_PER_SC + i
        base_a = f * FIELD_DIM - 32 * (f % 4)
        my_base = pl.multiple_of(base_a + tid * _CW, 128)
        buf = 0

        tail = (f == NUM_FIELDS - 1) & (tid == 15)
        cw_eff = jnp.where(tail, _AW + _TBW, _CW)

        cpA = pltpu.async_copy(embT.at[:, pl.ds(my_base, _AW)], chunkA, sem)

        @pl.when(~tail)
        def _():
            pltpu.async_copy(embT.at[:, pl.ds(my_base + _AW, _BW)],
                             chunkB, sem2)

        @pl.when(tail)
        def _():
            pltpu.async_copy(
                embT.at[:, pl.ds(TOTAL_VOCAB - 64 - _TBW, _TBW)],
                chunkB.at[:, pl.ds(0, _TBW)], sem2)

        pltpu.sync_copy(idx1d.at[pl.ds(f * BATCH, BATCH)], idxf)

        def scan_step(k4, off):
            for u in range(4):
                k = k4 * 4 + u
                iv = idxf[pl.ds(k * 16, 16)]
                vl = iv - my_base
                m = (vl >= 0) & (vl < cw_eff)
                nh = jnp.sum(m.astype(jnp.int32))
                b_vec = k * 16 + d16

                @pl.when(nh > 0)
                def _(off=off, vl=vl, m=m, b_vec=b_vec):
                    plsc.store_compressed(hitvlA.at[pl.ds(off, 16)], vl,
                                          mask=m)
                    plsc.store_compressed(hitbA.at[pl.ds(off, 16)], b_vec,
                                          mask=m)

                off = off + nh
            return off

        n_tot = lax.fori_loop(0, _NVREG // 4, scan_step, 0)

        def run_hits(srcA, srcB, hvl, hb, n_hits):
            hvl[pl.ds(n_hits, 16)] = jnp.zeros((16,), jnp.int32)
            hb[pl.ds(n_hits, 16)] = jnp.full((16,), BATCH, jnp.int32)
            ngrp = (n_hits + 15) // 16

            def grp_step(g, carry):
                vb = (g % 2) * 256
                rb = (g % 2) * 2
                vl16 = hvl[pl.ds(g * 16, 16)]
                b16 = hb[pl.ds(g * 16, 16)]
                bad = b16 >= BATCH
                inA = vl16 < _AW
                vlA = jnp.minimum(vl16, _AW - 1)
                vlB = jnp.maximum(vl16 - _AW, 0)
                btile = ((b16 >> 7) << 10) + (b16 & 127)

                @pl.when(g >= 2)
                def _():
                    pltpu.make_async_copy(acc_v.at[pl.ds(vb, 128)],
                                          shared.at[acc_p2.at[rb]],
                                          semg).wait()
                    pltpu.make_async_copy(acc_v.at[pl.ds(vb + 128, 128)],
                                          shared.at[acc_p2.at[rb + 1]],
                                          semg).wait()

                for d in range(16):
                    dv = jnp.full((16,), d, jnp.int32)
                    valsA = plsc.load_gather(srcA, [dv, vlA])
                    valsB = plsc.load_gather(srcB, [dv, vlB])
                    vals = jnp.where(inA, valsA, valsB)
                    cpos = (d // 8) * 32768 + (d % 8) * 128
                    pos = jnp.where(bad, _SLOTS + (d * 16) % 128 + d16,
                                    btile + cpos)
                    acc_v[pl.ds(vb + d * 16, 16)] = vals
                    acc_p2[rb + d // 8, pl.ds((d % 8) * 16, 16)] = pos + buf
                pltpu.async_copy(acc_v.at[pl.ds(vb, 128)],
                                 shared.at[acc_p2.at[rb]], semg)
                pltpu.async_copy(acc_v.at[pl.ds(vb + 128, 128)],
                                 shared.at[acc_p2.at[rb + 1]], semg)
                return carry

            lax.fori_loop(0, ngrp, grp_step, 0)

            def drain_step(g, carry):
                vb = (g % 2) * 256
                rb = (g % 2) * 2
                pltpu.make_async_copy(acc_v.at[pl.ds(vb, 128)],
                                      shared.at[acc_p2.at[rb]], semg).wait()
                pltpu.make_async_copy(acc_v.at[pl.ds(vb + 128, 128)],
                                      shared.at[acc_p2.at[rb + 1]],
                                      semg).wait()
                return carry

            lax.fori_loop(jnp.maximum(ngrp - 2, 0), ngrp, drain_step, 0)

        cpA.wait()

        @pl.when(~tail)
        def _():
            pltpu.make_async_copy(embT.at[:, pl.ds(0, _BW)], chunkB,
                                  sem2).wait()

        @pl.when(tail)
        def _():
            pltpu.make_async_copy(embT.at[:, pl.ds(0, _TBW)],
                                  chunkB.at[:, pl.ds(0, _TBW)], sem2).wait()

        run_hits(chunkA, chunkB, hitvlA, hitbA, n_tot)

        @pl.when(tail)
        def _():
            def tail_scan(k, off):
                iv = idxf[pl.ds(k * 16, 16)]
                vt = iv - (TOTAL_VOCAB - 64)
                mt = (vt >= 0) & (vt < 64)
                nt = jnp.sum(mt.astype(jnp.int32))

                @pl.when(nt > 0)
                def _():
                    plsc.store_compressed(hitvlA.at[pl.ds(off, 16)], vt,
                                          mask=mt)
                    plsc.store_compressed(hitbA.at[pl.ds(off, 16)],
                                          k * 16 + d16, mask=mt)

                return off + nt

            n_t = lax.fori_loop(0, _NVREG, tail_scan, 0)
            run_hits(chunk_tail, chunk_tail, hitvlA, hitbA, n_t)

        plsc.subcore_barrier()
        pltpu.sync_copy(
            shared.at[pl.ds(tid * (_SLOTS // 16), _SLOTS // 16)],
            out1d.at[pl.ds(f * _SLOTS + tid * (_SLOTS // 16), _SLOTS // 16)])
        plsc.subcore_barrier()
        return carry

    lax.fori_loop(0, _FIELDS_PER_SC, field_step, 0)


def _sc_gather(embT, idx1d):
    mesh = plsc.VectorSubcoreMesh(core_axis_name="c", subcore_axis_name="s")
    fn = pl.kernel(
        _sc_body,
        out_type=[
            jax.ShapeDtypeStruct((NUM_FIELDS * _SLOTS,), jnp.float32),
        ],
        mesh=mesh,
        scratch_types=[
            pltpu.VMEM((16, 3200), jnp.float32),
            pltpu.VMEM((16, 3072), jnp.float32),
            pltpu.VMEM((16, 64), jnp.float32),
            pltpu.VMEM((BATCH,), jnp.int32),
            pltpu.VMEM((_HITCAP,), jnp.int32),
            pltpu.VMEM((_HITCAP,), jnp.int32),
            pltpu.VMEM((512,), jnp.float32),
            pltpu.VMEM((4, 128), jnp.int32),
            pltpu.VMEM_SHARED((_SLAB_ALLOC,), jnp.float32),
            pltpu.SemaphoreType.DMA,
            pltpu.SemaphoreType.DMA,
            pltpu.SemaphoreType.DMA,
        ],
        compiler_params=pltpu.CompilerParams(use_tc_tiling_on_sc=True,
                                             needs_layout_passes=False,
                                             skip_device_barrier=True),
    )
    return fn(embT, idx1d)


_NPW = (BATCH * NUM_FIELDS) // 32


def _sc_lin_body(idx1d, lin1d, outlin, idxw, linw, sem):
    wid = lax.axis_index("s") * 2 + lax.axis_index("c")
    base = wid * _NPW
    pltpu.sync_copy(idx1d.at[pl.ds(base, _NPW)], idxw)

    def fire(j, carry):
        pltpu.async_copy(lin1d.at[idxw.at[pl.ds(j * 128, 128)]],
                         linw.at[pl.ds(j * 128, 128)], sem)
        return carry

    lax.fori_loop(0, _NPW // 128, fire, 0)

    def drain(j, carry):
        pltpu.make_async_copy(lin1d.at[pl.ds(0, 128)],
                              linw.at[pl.ds(j * 128, 128)], sem).wait()
        return carry

    lax.fori_loop(0, _NPW // 128, drain, 0)
    pltpu.sync_copy(linw, outlin.at[pl.ds(base, _NPW)])


def _sc_lin(idx1d, lin1d):
    mesh = plsc.VectorSubcoreMesh(core_axis_name="c", subcore_axis_name="s")
    fn = pl.kernel(
        _sc_lin_body,
        out_type=[jax.ShapeDtypeStruct((NUM_FIELDS * BATCH,), jnp.float32)],
        mesh=mesh,
        scratch_types=[
            pltpu.VMEM((_NPW,), jnp.int32),
            pltpu.VMEM((_NPW,), jnp.float32),
            pltpu.SemaphoreType.DMA,
        ],
        compiler_params=pltpu.CompilerParams(use_tc_tiling_on_sc=True,
                                             needs_layout_passes=False,
                                             skip_device_barrier=True),
    )
    return fn(idx1d, lin1d)


def _dotT(a, b):
    return lax.dot_general(a, b, (((0,), (0,)), ((), ())),
                           preferred_element_type=jnp.float32)


def _bn_prelu_T(h, g, be, a):
    mean = jnp.mean(h, axis=1, keepdims=True)
    var = jnp.mean(h * h, axis=1, keepdims=True) - mean * mean
    h = (h - mean) * lax.rsqrt(var + 1e-5) * g + be
    return jnp.maximum(h, 0.0) + a * jnp.minimum(h, 0.0)


def _tc_body(emb_ref, lin_ref, bias_ref, W1_ref, b1_ref, g1_ref, be1_ref,
             a1_ref, W2_ref, b2_ref, g2_ref, be2_ref, a2_ref, W3_ref,
             b3_ref, out_ref):
    emb = emb_ref[...]

    row = lax.broadcasted_iota(jnp.int32, (EMBED_OUT, EMBED_DIM), 0)
    col = lax.broadcasted_iota(jnp.int32, (EMBED_OUT, EMBED_DIM), 1)
    S = (row % EMBED_DIM == col).astype(jnp.float32)

    sum_f = _dotT(S, emb)
    sum_sq = _dotT(S, emb * emb)
    fm = 0.5 * jnp.sum(sum_f * sum_f - sum_sq, axis=0, keepdims=True)

    linear = jnp.sum(lin_ref[...], axis=0, keepdims=True) + bias_ref[0, 0]

    h = _dotT(W1_ref[...], emb) + b1_ref[...]
    h = _bn_prelu_T(h, g1_ref[...], be1_ref[...], a1_ref[0, 0])
    h = _dotT(W2_ref[...], h) + b2_ref[...]
    h = _bn_prelu_T(h, g2_ref[...], be2_ref[...], a2_ref[0, 0])
    mlp = _dotT(W3_ref[...], h) + b3_ref[0, 0]

    out_ref[...] = jnp.maximum(linear + fm + mlp, 0.0)


def _tc_dense(emb2, lin2, bias, W1, b1, g1, be1, a1, W2, b2, g2, be2, a2,
              W3, b3):
    return pl.pallas_call(
        _tc_body,
        out_shape=jax.ShapeDtypeStruct((1, BATCH), jnp.float32),
    )(emb2, lin2, bias.reshape(1, 1), W1, b1.reshape(-1, 1),
      g1.reshape(-1, 1), be1.reshape(-1, 1), a1.reshape(1, 1), W2,
      b2.reshape(-1, 1), g2.reshape(-1, 1), be2.reshape(-1, 1),
      a2.reshape(1, 1), W3, b3.reshape(1, 1))


_OFFSETS = np.arange(NUM_FIELDS, dtype=np.int32) * FIELD_DIM


def kernel(x, emb_table, lin_table, bias, W1, b1, g1, be1, a1, W2, b2, g2,
           be2, a2, W3, b3):
    idx1d = (x + jnp.asarray(_OFFSETS)[None, :]).T.reshape(-1)
    embT = emb_table.T
    lin1d = lin_table.T.reshape(-1)
    out1d, = _sc_gather(embT, idx1d)
    outlin, = _sc_lin(idx1d, lin1d)
    emb2 = (out1d.reshape(EMBED_OUT // 8, BATCH // 128, 8, 128)
            .transpose(0, 2, 1, 3).reshape(EMBED_OUT, BATCH))
    lin2 = outlin.reshape(NUM_FIELDS, BATCH)
    res = _tc_dense(emb2, lin2, bias, W1, b1, g1, be1, a1, W2, b2, g2, be2,
                    a2, W3, b3)
    return res.reshape(BATCH, 1)


# speedup vs baseline: 19.0399x; 1.1098x over previous
import functools

import jax
import jax.numpy as jnp
import numpy as np
from jax import lax
from jax.experimental import pallas as pl
from jax.experimental.pallas import tpu as pltpu
from jax.experimental.pallas import tpu_sc as plsc

NUM_FIELDS = 26
FIELD_DIM = 100000
EMBED_DIM = 16
TOTAL_VOCAB = NUM_FIELDS * FIELD_DIM
EMBED_OUT = NUM_FIELDS * EMBED_DIM
BATCH = 4096

_CW = 6272
_SLABW = 16 * _CW
_NVREG = BATCH // 16
_SLOTS = EMBED_DIM * BATCH
_SLAB_ALLOC = _SLOTS + 128
_FIELDS_PER_SC = NUM_FIELDS // 2
_HITCAP = BATCH + 16


def _sc_body(embT, idx1d, out1d,
             chunkA, chunkB, chunk_tail, idxf, hitvlA, hitbA,
             acc_v, acc_p2, shared, sem, sem2, semI, semg):
    cid = lax.axis_index("c")
    tid = lax.axis_index("s")
    d16 = lax.iota(jnp.int32, 16)
    _AW = 3200
    _BW = 3072
    _TBW = 2688

    pltpu.sync_copy(embT.at[:, pl.ds(TOTAL_VOCAB - 64, 64)], chunk_tail)

    def slab_base(f):
        return pl.multiple_of(f * FIELD_DIM - 32 * (f % 4) + tid * _CW, 128)

    def issue_stage(f, is_tail):
        mb = slab_base(f)
        pltpu.async_copy(embT.at[:, pl.ds(mb, _AW)], chunkA, sem)

        @pl.when(~is_tail)
        def _():
            pltpu.async_copy(embT.at[:, pl.ds(mb + _AW, _BW)], chunkB, sem2)

        @pl.when(is_tail)
        def _():
            pltpu.async_copy(
                embT.at[:, pl.ds(TOTAL_VOCAB - 64 - _TBW, _TBW)],
                chunkB.at[:, pl.ds(0, _TBW)], sem2)

    def issue_idx(f, slot):
        pltpu.async_copy(idx1d.at[pl.ds(f * BATCH, BATCH)],
                         idxf.at[pl.ds(slot * BATCH, BATCH)], semI)

    f0 = cid * _FIELDS_PER_SC
    issue_idx(f0, 0)
    issue_stage(f0, (f0 == NUM_FIELDS - 1) & (tid == 15))

    def field_step(i, carry):
        f = cid * _FIELDS_PER_SC + i
        ib = i % 2
        my_base = slab_base(f)
        buf = 0

        tail = (f == NUM_FIELDS - 1) & (tid == 15)
        cw_eff = jnp.where(tail, _AW + _TBW, _CW)

        pltpu.make_async_copy(idx1d.at[pl.ds(0, BATCH)],
                              idxf.at[pl.ds(ib * BATCH, BATCH)], semI).wait()

        def scan_step(k4, off):
            for u in range(4):
                k = k4 * 4 + u
                iv = idxf[pl.ds(ib * BATCH + k * 16, 16)]
                vl = iv - my_base
                m = (vl >= 0) & (vl < cw_eff)
                nh = jnp.sum(m.astype(jnp.int32))
                b_vec = k * 16 + d16

                @pl.when(nh > 0)
                def _(off=off, vl=vl, m=m, b_vec=b_vec):
                    plsc.store_compressed(hitvlA.at[pl.ds(off, 16)], vl,
                                          mask=m)
                    plsc.store_compressed(hitbA.at[pl.ds(off, 16)], b_vec,
                                          mask=m)

                off = off + nh
            return off

        n_tot = lax.fori_loop(0, _NVREG // 4, scan_step, 0)

        def run_hits(srcA, srcB, hvl, hb, n_hits):
            hvl[pl.ds(n_hits, 16)] = jnp.zeros((16,), jnp.int32)
            hb[pl.ds(n_hits, 16)] = jnp.full((16,), BATCH, jnp.int32)
            ngrp = (n_hits + 15) // 16

            def grp_step(g, carry):
                vb = (g % 2) * 256
                rb = (g % 2) * 2
                vl16 = hvl[pl.ds(g * 16, 16)]
                b16 = hb[pl.ds(g * 16, 16)]
                bad = b16 >= BATCH
                inA = vl16 < _AW
                vlA = jnp.minimum(vl16, _AW - 1)
                vlB = jnp.maximum(vl16 - _AW, 0)
                btile = ((b16 >> 7) << 10) + (b16 & 127)

                @pl.when(g >= 2)
                def _():
                    pltpu.make_async_copy(acc_v.at[pl.ds(vb, 128)],
                                          shared.at[acc_p2.at[rb]],
                                          semg).wait()
                    pltpu.make_async_copy(acc_v.at[pl.ds(vb + 128, 128)],
                                          shared.at[acc_p2.at[rb + 1]],
                                          semg).wait()

                for d in range(16):
                    dv = jnp.full((16,), d, jnp.int32)
                    valsA = plsc.load_gather(srcA, [dv, vlA])
                    valsB = plsc.load_gather(srcB, [dv, vlB])
                    vals = jnp.where(inA, valsA, valsB)
                    cpos = (d // 8) * 32768 + (d % 8) * 128
                    pos = jnp.where(bad, _SLOTS + (d * 16) % 128 + d16,
                                    btile + cpos)
                    acc_v[pl.ds(vb + d * 16, 16)] = vals
                    acc_p2[rb + d // 8, pl.ds((d % 8) * 16, 16)] = pos + buf
                pltpu.async_copy(acc_v.at[pl.ds(vb, 128)],
                                 shared.at[acc_p2.at[rb]], semg)
                pltpu.async_copy(acc_v.at[pl.ds(vb + 128, 128)],
                                 shared.at[acc_p2.at[rb + 1]], semg)
                return carry

            lax.fori_loop(0, ngrp, grp_step, 0)

            def drain_step(g, carry):
                vb = (g % 2) * 256
                rb = (g % 2) * 2
                pltpu.make_async_copy(acc_v.at[pl.ds(vb, 128)],
                                      shared.at[acc_p2.at[rb]], semg).wait()
                pltpu.make_async_copy(acc_v.at[pl.ds(vb + 128, 128)],
                                      shared.at[acc_p2.at[rb + 1]],
                                      semg).wait()
                return carry

            lax.fori_loop(jnp.maximum(ngrp - 2, 0), ngrp, drain_step, 0)

        @pl.when(i < _FIELDS_PER_SC - 1)
        def _():
            issue_idx(f + 1, 1 - ib)

        pltpu.make_async_copy(embT.at[:, pl.ds(0, _AW)], chunkA, sem).wait()

        @pl.when(~tail)
        def _():
            pltpu.make_async_copy(embT.at[:, pl.ds(0, _BW)], chunkB,
                                  sem2).wait()

        @pl.when(tail)
        def _():
            pltpu.make_async_copy(embT.at[:, pl.ds(0, _TBW)],
                                  chunkB.at[:, pl.ds(0, _TBW)], sem2).wait()

        run_hits(chunkA, chunkB, hitvlA, hitbA, n_tot)

        @pl.when(tail)
        def _():
            def tail_scan(k, off):
                iv = idxf[pl.ds(ib * BATCH + k * 16, 16)]
                vt = iv - (TOTAL_VOCAB - 64)
                mt = (vt >= 0) & (vt < 64)
                nt = jnp.sum(mt.astype(jnp.int32))

                @pl.when(nt > 0)
                def _():
                    plsc.store_compressed(hitvlA.at[pl.ds(off, 16)], vt,
                                          mask=mt)
                    plsc.store_compressed(hitbA.at[pl.ds(off, 16)],
                                          k * 16 + d16, mask=mt)

                return off + nt

            n_t = lax.fori_loop(0, _NVREG, tail_scan, 0)
            run_hits(chunk_tail, chunk_tail, hitvlA, hitbA, n_t)

        @pl.when(i < _FIELDS_PER_SC - 1)
        def _():
            nf = f + 1
            issue_stage(nf, (nf == NUM_FIELDS - 1) & (tid == 15))

        plsc.subcore_barrier()
        pltpu.sync_copy(
            shared.at[pl.ds(tid * (_SLOTS // 16), _SLOTS // 16)],
            out1d.at[pl.ds(f * _SLOTS + tid * (_SLOTS // 16), _SLOTS // 16)])
        plsc.subcore_barrier()
        return carry

    lax.fori_loop(0, _FIELDS_PER_SC, field_step, 0)


def _sc_gather(embT, idx1d):
    mesh = plsc.VectorSubcoreMesh(core_axis_name="c", subcore_axis_name="s")
    fn = pl.kernel(
        _sc_body,
        out_type=[
            jax.ShapeDtypeStruct((NUM_FIELDS * _SLOTS,), jnp.float32),
        ],
        mesh=mesh,
        scratch_types=[
            pltpu.VMEM((16, 3200), jnp.float32),
            pltpu.VMEM((16, 3072), jnp.float32),
            pltpu.VMEM((16, 64), jnp.float32),
            pltpu.VMEM((2 * BATCH,), jnp.int32),
            pltpu.VMEM((_HITCAP,), jnp.int32),
            pltpu.VMEM((_HITCAP,), jnp.int32),
            pltpu.VMEM((512,), jnp.float32),
            pltpu.VMEM((4, 128), jnp.int32),
            pltpu.VMEM_SHARED((_SLAB_ALLOC,), jnp.float32),
            pltpu.SemaphoreType.DMA,
            pltpu.SemaphoreType.DMA,
            pltpu.SemaphoreType.DMA,
            pltpu.SemaphoreType.DMA,
        ],
        compiler_params=pltpu.CompilerParams(use_tc_tiling_on_sc=True,
                                             needs_layout_passes=False,
                                             skip_device_barrier=True),
    )
    return fn(embT, idx1d)


_NPW = (BATCH * NUM_FIELDS) // 32


def _sc_lin_body(idx1d, lin1d, outlin, idxw, linw, sem):
    wid = lax.axis_index("s") * 2 + lax.axis_index("c")
    base = wid * _NPW
    pltpu.sync_copy(idx1d.at[pl.ds(base, _NPW)], idxw)

    def fire(j, carry):
        pltpu.async_copy(lin1d.at[idxw.at[pl.ds(j * 128, 128)]],
                         linw.at[pl.ds(j * 128, 128)], sem)
        return carry

    lax.fori_loop(0, _NPW // 128, fire, 0)

    def drain(j, carry):
        pltpu.make_async_copy(lin1d.at[pl.ds(0, 128)],
                              linw.at[pl.ds(j * 128, 128)], sem).wait()
        return carry

    lax.fori_loop(0, _NPW // 128, drain, 0)
    pltpu.sync_copy(linw, outlin.at[pl.ds(base, _NPW)])


def _sc_lin(idx1d, lin1d):
    mesh = plsc.VectorSubcoreMesh(core_axis_name="c", subcore_axis_name="s")
    fn = pl.kernel(
        _sc_lin_body,
        out_type=[jax.ShapeDtypeStruct((NUM_FIELDS * BATCH,), jnp.float32)],
        mesh=mesh,
        scratch_types=[
            pltpu.VMEM((_NPW,), jnp.int32),
            pltpu.VMEM((_NPW,), jnp.float32),
            pltpu.SemaphoreType.DMA,
        ],
        compiler_params=pltpu.CompilerParams(use_tc_tiling_on_sc=True,
                                             needs_layout_passes=False,
                                             skip_device_barrier=True),
    )
    return fn(idx1d, lin1d)


def _dotT(a, b):
    return lax.dot_general(a, b, (((0,), (0,)), ((), ())),
                           preferred_element_type=jnp.float32)


def _bn_prelu_T(h, g, be, a):
    mean = jnp.mean(h, axis=1, keepdims=True)
    var = jnp.mean(h * h, axis=1, keepdims=True) - mean * mean
    h = (h - mean) * lax.rsqrt(var + 1e-5) * g + be
    return jnp.maximum(h, 0.0) + a * jnp.minimum(h, 0.0)


def _tc_body(emb_ref, lin_ref, bias_ref, W1_ref, b1_ref, g1_ref, be1_ref,
             a1_ref, W2_ref, b2_ref, g2_ref, be2_ref, a2_ref, W3_ref,
             b3_ref, out_ref):
    emb = emb_ref[...]

    row = lax.broadcasted_iota(jnp.int32, (EMBED_OUT, EMBED_DIM), 0)
    col = lax.broadcasted_iota(jnp.int32, (EMBED_OUT, EMBED_DIM), 1)
    S = (row % EMBED_DIM == col).astype(jnp.float32)

    sum_f = _dotT(S, emb)
    sum_sq = _dotT(S, emb * emb)
    fm = 0.5 * jnp.sum(sum_f * sum_f - sum_sq, axis=0, keepdims=True)

    linear = jnp.sum(lin_ref[...], axis=0, keepdims=True) + bias_ref[0, 0]

    h = _dotT(W1_ref[...], emb) + b1_ref[...]
    h = _bn_prelu_T(h, g1_ref[...], be1_ref[...], a1_ref[0, 0])
    h = _dotT(W2_ref[...], h) + b2_ref[...]
    h = _bn_prelu_T(h, g2_ref[...], be2_ref[...], a2_ref[0, 0])
    mlp = _dotT(W3_ref[...], h) + b3_ref[0, 0]

    out_ref[...] = jnp.maximum(linear + fm + mlp, 0.0)


def _tc_dense(emb2, lin2, bias, W1, b1, g1, be1, a1, W2, b2, g2, be2, a2,
              W3, b3):
    return pl.pallas_call(
        _tc_body,
        out_shape=jax.ShapeDtypeStruct((1, BATCH), jnp.float32),
    )(emb2, lin2, bias.reshape(1, 1), W1, b1.reshape(-1, 1),
      g1.reshape(-1, 1), be1.reshape(-1, 1), a1.reshape(1, 1), W2,
      b2.reshape(-1, 1), g2.reshape(-1, 1), be2.reshape(-1, 1),
      a2.reshape(1, 1), W3, b3.reshape(1, 1))


_OFFSETS = np.arange(NUM_FIELDS, dtype=np.int32) * FIELD_DIM


def kernel(x, emb_table, lin_table, bias, W1, b1, g1, be1, a1, W2, b2, g2,
           be2, a2, W3, b3):
    idx1d = (x + jnp.asarray(_OFFSETS)[None, :]).T.reshape(-1)
    embT = emb_table.T
    lin1d = lin_table.T.reshape(-1)
    out1d, = _sc_gather(embT, idx1d)
    outlin, = _sc_lin(idx1d, lin1d)
    emb2 = (out1d.reshape(EMBED_OUT // 8, BATCH // 128, 8, 128)
            .transpose(0, 2, 1, 3).reshape(EMBED_OUT, BATCH))
    lin2 = outlin.reshape(NUM_FIELDS, BATCH)
    res = _tc_dense(emb2, lin2, bias, W1, b1, g1, be1, a1, W2, b2, g2, be2,
                    a2, W3, b3)
    return res.reshape(BATCH, 1)
